# baseline (speedup 1.0000x reference)
import functools

import jax
import jax.numpy as jnp
from jax import lax
from jax.experimental import pallas as pl
from jax.experimental.pallas import tpu as pltpu
from jax.experimental.pallas import tpu_sc as plsc

VOCAB = 1000000
DIM = 64
BATCH = 16384
CTX = 20

NC = 2
NS = 16
NW = NC * NS
BPW = BATCH // NW
STEP = 32
NSTEP = BPW // STEP
NG = CTX * STEP // 128
NK = DIM // 16


def _body(cs_hbm, os_hbm, word_hbm, bkp_hbm, out_hbm,
          idx_os, idx_cs, idx_csh, idx_steps, bufs, cbuf, prow, ysig, sem):
    wid = lax.axis_index("s") * NC + lax.axis_index("c")
    base = wid * BPW

    pltpu.sync_copy(cs_hbm.at[pl.ds(base, BPW)], idx_cs.at[pl.ds(0, BPW)])
    for c in range(CTX):
        pltpu.sync_copy(os_hbm.at[c, pl.ds(base, BPW)],
                        idx_os.at[c, pl.ds(0, BPW)])

    for q in range(BPW // 16):
        idx_csh[pl.ds(q * 16, 16)] = idx_cs[pl.ds(q * 16, 16)] >> 1
    for s in range(NSTEP):
        for c in range(CTX):
            for h in range(STEP // 16):
                v = idx_os[c, pl.ds(s * STEP + h * 16, 16)]
                idx_steps[s, pl.ds(c * STEP + h * 16, 16)] = v >> 1

    lane = lax.iota(jnp.int32, 16)

    def step(si, carry):
        sbase = si * STEP
        copies = []
        for g in range(NG):
            cp = pltpu.make_async_copy(
                word_hbm.at[idx_steps.at[si, pl.ds(g * 128, 128)]],
                bufs.at[pl.ds(g * 128, 128)], sem)
            cp.start()
            copies.append(cp)
        cpc = pltpu.make_async_copy(
            bkp_hbm.at[idx_csh.at[pl.ds(sbase, STEP)]], cbuf, sem)
        cpc.start()
        for cp in copies:
            cp.wait()
        cpc.wait()

        def row(r, rcarry):
            pr = jnp.zeros((16,), jnp.float32)
            cpar = (idx_cs[pl.ds(sbase + r, 16)][0] & 1) * 64
            pars = [(idx_os[c, pl.ds(sbase + r, 16)][0] & 1) * 64
                    for c in range(CTX)]
            for k in range(NK):
                a = bufs[r, pl.ds(pars[0] + k * 16, 16)]
                for c in range(1, CTX):
                    a = a + bufs[c * STEP + r, pl.ds(pars[c] + k * 16, 16)]
                pr = pr + a * cbuf[r, pl.ds(cpar + k * 16, 16)]
            prow[r] = pr * (1.0 / CTX)
            return rcarry

        lax.fori_loop(0, STEP, row, 0, unroll=2)

        for g in range(STEP // 16):
            y = jnp.zeros((16,), jnp.float32)
            for l in range(16):
                s = jnp.sum(prow[g * 16 + l])
                y = jnp.where(lane == l, s, y)
            ysig[pl.ds(sbase + g * 16, 16)] = 1.0 / (1.0 + jnp.exp(-y))
        return carry

    lax.fori_loop(0, NSTEP, step, 0)

    pltpu.sync_copy(ysig, out_hbm.at[pl.ds(base, BPW)])


@jax.jit
def _cbow(cs, os, word_embs, bkp_word_embs):
    w2 = word_embs.reshape(VOCAB // 2, 2 * DIM)
    b2 = bkp_word_embs.reshape(VOCAB // 2, 2 * DIM)
    mesh = plsc.VectorSubcoreMesh(core_axis_name="c", subcore_axis_name="s")
    f = pl.kernel(
        _body,
        out_type=jax.ShapeDtypeStruct((BATCH,), jnp.float32),
        mesh=mesh,
        compiler_params=pltpu.CompilerParams(needs_layout_passes=False),
        scratch_types=[
            pltpu.VMEM((CTX, BPW + 16), jnp.int32),
            pltpu.VMEM((BPW + 16,), jnp.int32),
            pltpu.VMEM((BPW,), jnp.int32),
            pltpu.VMEM((NSTEP, CTX * STEP), jnp.int32),
            pltpu.VMEM((CTX * STEP, 2 * DIM), jnp.float32),
            pltpu.VMEM((STEP, 2 * DIM), jnp.float32),
            pltpu.VMEM((STEP, 16), jnp.float32),
            pltpu.VMEM((BPW,), jnp.float32),
            pltpu.SemaphoreType.DMA,
        ],
    )
    return f(cs, os, w2, b2)


def kernel(cs, os, word_embs, bkp_word_embs):
    return _cbow(cs, os, word_embs, bkp_word_embs)
